# baseline (speedup 1.0000x reference)
import jax
import jax.numpy as jnp
from jax.experimental import pallas as pl
from jax.experimental.pallas import tpu as pltpu

_IX = 5
_BATCH = 16384
_HIST = 200

_KS0 = 0
_KS1 = 42
_KS2 = (0x1BD11BDA ^ _KS0 ^ _KS1) & 0xFFFFFFFF

_ROT_A = (13, 15, 26, 6)
_ROT_B = (17, 29, 16, 24)
_INJ = ((_KS1, _KS2 + 1), (_KS2, _KS0 + 2), (_KS0, _KS1 + 3),
        (_KS1, _KS2 + 4), (_KS2, _KS0 + 5))
_ROTS = (_ROT_A, _ROT_B, _ROT_A, _ROT_B, _ROT_A)

_COLS_PER_BLOCK = 512
_GRID = _BATCH // _COLS_PER_BLOCK


def _rotl(v, d):
    return (v << jnp.uint32(d)) | (v >> jnp.uint32(32 - d))


def _hash_keep(flat):
    x1 = flat + jnp.uint32(_KS1)
    x0 = x1
    x1 = x0 ^ _rotl(x1, _ROT_A[0])
    for d in _ROT_A[1:]:
        x0 = x0 + x1
        x1 = x0 ^ _rotl(x1, d)
    x0 = x0 + jnp.uint32(_INJ[0][0])
    x1 = x1 + jnp.uint32(_INJ[0][1])
    for g in range(1, 5):
        for d in _ROTS[g]:
            x0 = x0 + x1
            x1 = x0 ^ _rotl(x1, d)
        x0 = x0 + jnp.uint32(_INJ[g][0])
        x1 = x1 + jnp.uint32(_INJ[g][1])
    return (x0 ^ x1).astype(jnp.int32) >= 0


def _replace_kernel(x_ref, o_ref):
    i = pl.program_id(0)
    base = (i * _COLS_PER_BLOCK * _HIST).astype(jnp.uint32)
    sub = jax.lax.broadcasted_iota(jnp.uint32, (_HIST, _COLS_PER_BLOCK), 0)
    lane = jax.lax.broadcasted_iota(jnp.uint32, (_HIST, _COLS_PER_BLOCK), 1)
    flat = base + lane * jnp.uint32(_HIST) + sub
    keep = _hash_keep(flat)
    o_ref[...] = jnp.where(keep, x_ref[...], jnp.int32(_IX))


def kernel(x):
    xt = x.T
    xt = pltpu.with_memory_space_constraint(xt, pltpu.MemorySpace.HBM)
    out_t = pl.pallas_call(
        _replace_kernel,
        grid=(_GRID,),
        in_specs=[pl.BlockSpec((_HIST, _COLS_PER_BLOCK), lambda i: (0, i))],
        out_specs=pl.BlockSpec((_HIST, _COLS_PER_BLOCK), lambda i: (0, i)),
        out_shape=jax.ShapeDtypeStruct((_HIST, _BATCH), jnp.int32),
        compiler_params=pltpu.CompilerParams(
            dimension_semantics=("parallel",),
        ),
    )(xt)
    return out_t.T


# speedup vs baseline: 1.0103x; 1.0103x over previous
import jax
import jax.numpy as jnp
from jax.experimental import pallas as pl
from jax.experimental.pallas import tpu as pltpu

_IX = 5
_BATCH = 16384
_HIST = 200

_KS0 = 0
_KS1 = 42
_KS2 = (0x1BD11BDA ^ _KS0 ^ _KS1) & 0xFFFFFFFF

_ROT_A = (13, 15, 26, 6)
_ROT_B = (17, 29, 16, 24)
_INJ = ((_KS1, _KS2 + 1), (_KS2, _KS0 + 2), (_KS0, _KS1 + 3),
        (_KS1, _KS2 + 4), (_KS2, _KS0 + 5))
_ROTS = (_ROT_A, _ROT_B, _ROT_A, _ROT_B, _ROT_A)

_COLS_PER_BLOCK = 1024
_GRID = _BATCH // _COLS_PER_BLOCK


def _rotl(v, d):
    return (v << jnp.uint32(d)) | (v >> jnp.uint32(32 - d))


def _hash_keep(flat):
    x1 = flat + jnp.uint32(_KS1)
    x0 = x1
    x1 = x0 ^ _rotl(x1, _ROT_A[0])
    for d in _ROT_A[1:]:
        x0 = x0 + x1
        x1 = x0 ^ _rotl(x1, d)
    x0 = x0 + jnp.uint32(_INJ[0][0])
    x1 = x1 + jnp.uint32(_INJ[0][1])
    for g in range(1, 5):
        for d in _ROTS[g]:
            x0 = x0 + x1
            x1 = x0 ^ _rotl(x1, d)
        x0 = x0 + jnp.uint32(_INJ[g][0])
        x1 = x1 + jnp.uint32(_INJ[g][1])
    return (x0 ^ x1).astype(jnp.int32) >= 0


def _replace_kernel(x_ref, o_ref):
    i = pl.program_id(0)
    base = (i * _COLS_PER_BLOCK * _HIST).astype(jnp.uint32)
    sub = jax.lax.broadcasted_iota(jnp.uint32, (_HIST, _COLS_PER_BLOCK), 0)
    lane = jax.lax.broadcasted_iota(jnp.uint32, (_HIST, _COLS_PER_BLOCK), 1)
    flat = base + lane * jnp.uint32(_HIST) + sub
    keep = _hash_keep(flat)
    o_ref[...] = jnp.where(keep, x_ref[...], jnp.int32(_IX))


def kernel(x):
    xt = x.T
    xt = pltpu.with_memory_space_constraint(xt, pltpu.MemorySpace.HBM)
    out_t = pl.pallas_call(
        _replace_kernel,
        grid=(_GRID,),
        in_specs=[pl.BlockSpec((_HIST, _COLS_PER_BLOCK), lambda i: (0, i))],
        out_specs=pl.BlockSpec((_HIST, _COLS_PER_BLOCK), lambda i: (0, i)),
        out_shape=jax.ShapeDtypeStruct((_HIST, _BATCH), jnp.int32),
        compiler_params=pltpu.CompilerParams(
            dimension_semantics=("parallel",),
        ),
    )(xt)
    return out_t.T


# speedup vs baseline: 1.0109x; 1.0006x over previous
import jax
import jax.numpy as jnp
from jax.experimental import pallas as pl
from jax.experimental.pallas import tpu as pltpu

_IX = 5
_BATCH = 16384
_HIST = 200

_KS0 = 0
_KS1 = 42
_KS2 = (0x1BD11BDA ^ _KS0 ^ _KS1) & 0xFFFFFFFF

_ROT_A = (13, 15, 26, 6)
_ROT_B = (17, 29, 16, 24)
_INJ = ((_KS1, _KS2 + 1), (_KS2, _KS0 + 2), (_KS0, _KS1 + 3),
        (_KS1, _KS2 + 4), (_KS2, _KS0 + 5))
_ROTS = (_ROT_A, _ROT_B, _ROT_A, _ROT_B, _ROT_A)

_COLS_PER_BLOCK = 1024
_GRID = _BATCH // _COLS_PER_BLOCK


def _rotl(v, d):
    return (v << jnp.uint32(d)) | (v >> jnp.uint32(32 - d))


def _hash_keep(x1):
    x0 = x1
    x1 = x0 ^ _rotl(x1, _ROT_A[0])
    for d in _ROT_A[1:]:
        x0 = x0 + x1
        x1 = x0 ^ _rotl(x1, d)
    x0 = x0 + jnp.uint32(_INJ[0][0])
    x1 = x1 + jnp.uint32(_INJ[0][1])
    for g in range(1, 5):
        for d in _ROTS[g]:
            x0 = x0 + x1
            x1 = x0 ^ _rotl(x1, d)
        x0 = x0 + jnp.uint32(_INJ[g][0])
        x1 = x1 + jnp.uint32(_INJ[g][1])
    return (x0 ^ x1).astype(jnp.int32) >= 0


def _replace_kernel(x_ref, o_ref):
    i = pl.program_id(0)
    base = (i * _COLS_PER_BLOCK * _HIST + _KS1).astype(jnp.uint32)
    sub = jax.lax.broadcasted_iota(jnp.uint32, (_HIST, _COLS_PER_BLOCK), 0)
    lane = jax.lax.broadcasted_iota(jnp.uint32, (_HIST, _COLS_PER_BLOCK), 1)
    keep = _hash_keep(base + (lane * jnp.uint32(_HIST) + sub))
    o_ref[...] = jnp.where(keep, x_ref[...], jnp.int32(_IX))


def kernel(x):
    xt = x.T
    xt = pltpu.with_memory_space_constraint(xt, pltpu.MemorySpace.HBM)
    out_t = pl.pallas_call(
        _replace_kernel,
        grid=(_GRID,),
        in_specs=[pl.BlockSpec((_HIST, _COLS_PER_BLOCK), lambda i: (0, i))],
        out_specs=pl.BlockSpec((_HIST, _COLS_PER_BLOCK), lambda i: (0, i)),
        out_shape=jax.ShapeDtypeStruct((_HIST, _BATCH), jnp.int32),
        compiler_params=pltpu.CompilerParams(
            dimension_semantics=("parallel",),
        ),
    )(xt)
    return out_t.T
